# Initial kernel scaffold; baseline (speedup 1.0000x reference)
import jax
import jax.numpy as jnp
from jax.experimental import pallas as pl


def kernel(patches, memory_bank):
    raise NotImplementedError("write your pallas kernel here")


# speedup vs baseline: 1.0544x; 1.0544x over previous
import functools

import jax
import jax.numpy as jnp
from jax.experimental import pallas as pl
from jax.experimental.pallas import tpu as pltpu


def _knn_body(p_ref, m_ref, out_ref, minacc, *, nq, nk):
    i = pl.program_id(0)
    j = pl.program_id(1)
    p = p_ref[...]
    m = m_ref[...]
    dot = jax.lax.dot_general(
        p, m, (((1,), (1,)), ((), ())), preferred_element_type=jnp.float32
    )
    ones = jnp.ones((1, p.shape[1]), dtype=jnp.bfloat16)
    m2row = jax.lax.dot_general(
        ones, m * m, (((1,), (1,)), ((), ())), preferred_element_type=jnp.float32
    )
    tmin = jnp.min(m2row - 2.0 * dot, axis=1, keepdims=True)

    @pl.when(j == 0)
    def _():
        minacc[...] = tmin

    @pl.when(j > 0)
    def _():
        minacc[...] = jnp.minimum(minacc[...], tmin)

    @pl.when(j == nk - 1)
    def _():
        pf = p.astype(jnp.float32)
        p2 = jnp.sum(pf * pf, axis=1, keepdims=True)
        bmax = jnp.max(minacc[...] + p2)

        @pl.when(i == 0)
        def _():
            out_ref[0, 0] = bmax

        @pl.when(i > 0)
        def _():
            out_ref[0, 0] = jnp.maximum(out_ref[0, 0], bmax)

        @pl.when(i == nq - 1)
        def _():
            out_ref[0, 0] = jnp.sqrt(jnp.maximum(out_ref[0, 0], 1e-12))


def kernel(patches, memory_bank):
    q, d = patches.shape
    k, _ = memory_bank.shape
    bq = min(2048, q)
    bk = min(1024, k)
    nq, nk = q // bq, k // bk

    p16 = patches.astype(jnp.bfloat16)
    m16 = memory_bank.astype(jnp.bfloat16)

    out = pl.pallas_call(
        functools.partial(_knn_body, nq=nq, nk=nk),
        grid=(nq, nk),
        in_specs=[
            pl.BlockSpec((bq, d), lambda i, j: (i, 0)),
            pl.BlockSpec((bk, d), lambda i, j: (j, 0)),
        ],
        out_specs=pl.BlockSpec(
            (1, 1), lambda i, j: (0, 0), memory_space=pltpu.SMEM
        ),
        out_shape=jax.ShapeDtypeStruct((1, 1), jnp.float32),
        scratch_shapes=[pltpu.VMEM((bq, 1), jnp.float32)],
        compiler_params=pltpu.CompilerParams(
            dimension_semantics=("arbitrary", "arbitrary"),
        ),
    )(p16, m16)
    return out[0, 0]
